# baseline (speedup 1.0000x reference)
import jax
import jax.numpy as jnp
from jax.experimental import pallas as pl
from jax.experimental.pallas import tpu as pltpu

_B, _C, _H, _W = 128, 3, 224, 224
_IMGS = 8


def _gray_body(inds_ref, x_ref, o_ref):
    g = pl.program_id(0)
    for i in range(_IMGS):
        sel = inds_ref[g * _IMGS + i] != 0

        @pl.when(sel)
        def _(i=i):
            L = (x_ref[i, 0] * (299.0 / 1000.0)
                 + x_ref[i, 1] * (587.0 / 1000.0)
                 + x_ref[i, 2] * (114.0 / 1000.0))
            o_ref[i, 0] = L
            o_ref[i, 1] = L
            o_ref[i, 2] = L

        @pl.when(jnp.logical_not(sel))
        def _(i=i):
            o_ref[i] = x_ref[i]


def kernel(x, inds):
    out = pl.pallas_call(
        _gray_body,
        grid_spec=pltpu.PrefetchScalarGridSpec(
            num_scalar_prefetch=1,
            grid=(_B // _IMGS,),
            in_specs=[pl.BlockSpec((_IMGS, _C, _H, _W), lambda b, inds: (b, 0, 0, 0))],
            out_specs=pl.BlockSpec((_IMGS, _C, _H, _W), lambda b, inds: (b, 0, 0, 0)),
        ),
        out_shape=jax.ShapeDtypeStruct((_B, _C, _H, _W), jnp.float32),
        compiler_params=pltpu.CompilerParams(
            dimension_semantics=("parallel",),
        ),
    )(inds.astype(jnp.int32), x)
    return out


# speedup vs baseline: 1.0041x; 1.0041x over previous
import jax
import jax.numpy as jnp
from jax.experimental import pallas as pl
from jax.experimental.pallas import tpu as pltpu

_B, _C, _H, _W = 128, 3, 224, 224
_K = 12
_NTHREADS = 2


def _body(inds_ref, x_hbm, o_hbm, ibuf, obuf, isem, osem):
    def in_copy(b, s):
        pltpu.make_async_copy(x_hbm.at[b], ibuf.at[s], isem.at[s])

    for s in range(_K):
        pltpu.make_async_copy(x_hbm.at[s], ibuf.at[s], isem.at[s]).start(
            priority=s % 2)

    def step(b, carry):
        s = jax.lax.rem(b, _K)
        pltpu.make_async_copy(x_hbm.at[b], ibuf.at[s], isem.at[s]).wait()

        @pl.when(b >= _K)
        def _():
            pltpu.make_async_copy(obuf.at[s], o_hbm.at[b - _K], osem.at[s]).wait()

        sel = inds_ref[b] != 0

        @pl.when(sel)
        def _():
            L = (ibuf[s, 0] * (299.0 / 1000.0)
                 + ibuf[s, 1] * (587.0 / 1000.0)
                 + ibuf[s, 2] * (114.0 / 1000.0))
            obuf[s, 0] = L
            obuf[s, 1] = L
            obuf[s, 2] = L

        @pl.when(jnp.logical_not(sel))
        def _():
            obuf[s] = ibuf[s]

        for t in range(_NTHREADS):
            @pl.when(jax.lax.rem(s, _NTHREADS) == t)
            def _(t=t):
                pltpu.make_async_copy(obuf.at[s], o_hbm.at[b], osem.at[s]).start(
                    priority=t)

                @pl.when(b + _K < _B)
                def _():
                    pltpu.make_async_copy(
                        x_hbm.at[b + _K], ibuf.at[s], isem.at[s]).start(priority=t)

        return carry

    jax.lax.fori_loop(0, _B, step, 0)

    for s in range(_K):
        b = _B - _K + s
        pltpu.make_async_copy(obuf.at[s], o_hbm.at[b], osem.at[s]).wait()


def kernel(x, inds):
    out = pl.pallas_call(
        _body,
        grid_spec=pltpu.PrefetchScalarGridSpec(
            num_scalar_prefetch=1,
            grid=(1,),
            in_specs=[pl.BlockSpec(memory_space=pltpu.MemorySpace.HBM)],
            out_specs=pl.BlockSpec(memory_space=pltpu.MemorySpace.HBM),
            scratch_shapes=[
                pltpu.VMEM((_K, _C, _H, _W), jnp.float32),
                pltpu.VMEM((_K, _C, _H, _W), jnp.float32),
                pltpu.SemaphoreType.DMA((_K,)),
                pltpu.SemaphoreType.DMA((_K,)),
            ],
        ),
        out_shape=jax.ShapeDtypeStruct((_B, _C, _H, _W), jnp.float32),
    )(inds.astype(jnp.int32), x)
    return out
